# baseline (speedup 1.0000x reference)
import functools

import jax
import jax.numpy as jnp
from jax import lax
from jax.experimental import pallas as pl
from jax.experimental.pallas import tpu as pltpu
from jax.experimental.pallas import tpu_sc as plsc

L = 16
CW = 128
NCW = 245
ARR = 272
CAP = 16384
ICH = 2048
SROWS = 128
GPB = SROWS // L
DUMP = L


def _blk(ref, k):
    return ref[pl.ds(pl.multiple_of(k * L, L), L)]


@functools.lru_cache(maxsize=None)
def _make_kernel(total_rows, vocab):
    info = plsc.get_sparse_core_info()
    nc, ns = info.num_cores, info.num_subcores
    nw = nc * ns
    n_chunks = total_rows // ICH
    assert n_chunks * ICH == total_rows
    max_c = (vocab + CW - 1) // CW
    assert NCW * nw >= max_c
    last_off = (vocab // CW) * CW

    mesh = plsc.VectorSubcoreMesh(core_axis_name="c", subcore_axis_name="s")
    scratch = {
        "ids": pltpu.VMEM((2, ICH), jnp.int32),
        "hist": pltpu.VMEM((L, ARR), jnp.int32),
        "colsum": pltpu.VMEM((ARR,), jnp.int32),
        "colbase": pltpu.VMEM((ARR,), jnp.int32),
        "sidx": pltpu.VMEM((CAP,), jnp.int32),
        "spos": pltpu.VMEM((CAP,), jnp.int32),
        "colbuf": pltpu.VMEM((4, 64, CW), jnp.float32),
        "stage": pltpu.VMEM((2, SROWS, CW), jnp.float32),
        "pbuf": pltpu.VMEM((2, SROWS), jnp.int32),
        "sem_ids0": pltpu.SemaphoreType.DMA,
        "sem_ids1": pltpu.SemaphoreType.DMA,
        "sem_col0": pltpu.SemaphoreType.DMA,
        "sem_col1": pltpu.SemaphoreType.DMA,
        "sem_col2": pltpu.SemaphoreType.DMA,
        "sem_col3": pltpu.SemaphoreType.DMA,
        "sem_out0": pltpu.SemaphoreType.DMA,
        "sem_out1": pltpu.SemaphoreType.DMA,
    }

    @functools.partial(
        pl.kernel,
        mesh=mesh,
        out_type=jax.ShapeDtypeStruct((total_rows + DUMP, CW), jnp.float32),
        scratch_types=scratch,
        compiler_params=pltpu.CompilerParams(needs_layout_passes=False),
    )
    def gather_kernel(idx_hbm, tab_hbm, out_hbm, *, ids, hist, colsum,
                      colbase, sidx, spos, colbuf, stage, pbuf, sem_ids0,
                      sem_ids1, sem_col0, sem_col1, sem_col2, sem_col3,
                      sem_out0, sem_out1):
        wid = lax.axis_index("s") * nc + lax.axis_index("c")
        lo = wid * NCW
        hi = lo + NCW
        iota = lax.iota(jnp.int32, L)
        ones = jnp.ones((L,), jnp.int32)
        zeros16 = jnp.zeros((L,), jnp.int32)

        def bcast(x):
            return jnp.full((L,), 1, jnp.int32) * x

        def zrow(r, carry):
            rv = bcast(r)
            for k in range(ARR // L):
                plsc.store_scatter(hist, [rv, k * L + iota], zeros16)
            return carry

        lax.fori_loop(0, L, zrow, 0)

        id_sems = (sem_ids0, sem_ids1)

        def fire_ids(g, b):
            pltpu.make_async_copy(
                idx_hbm.at[pl.ds(pl.multiple_of(g * ICH, ICH), ICH)],
                ids.at[b], id_sems[b],
            ).start()

        def wait_ids(b):
            pltpu.make_async_copy(
                idx_hbm.at[pl.ds(0, ICH)], ids.at[b], id_sems[b]
            ).wait()

        def stream_pass(chunk_body):
            fire_ids(0, 0)
            fire_ids(1, 1)

            def pair(gp, carry):
                for b in range(2):
                    g = gp * 2 + b
                    wait_ids(b)
                    chunk_body(b, g)
                    fire_ids(g + 2, b)
                return carry

            lax.fori_loop(0, n_chunks // 2 - 1, pair, 0)
            for b in range(2):
                g = n_chunks - 2 + b
                wait_ids(b)
                chunk_body(b, g)

        def tok_stats(v):
            c = lax.shift_right_logical(v, 7)
            m = (c >= lo) & (c < hi)
            return m, c - lo

        def p1_chunk(b, g):
            def vec(i, carry2):
                v = ids[b, pl.ds(pl.multiple_of(i * L, L), L)]
                m, cl = tok_stats(v)
                plsc.addupdate_scatter(hist, [iota, cl], ones, mask=m)
                return carry2

            lax.fori_loop(0, ICH // L, vec, 0)

        stream_pass(p1_chunk)

        def mkbase(k, carry):
            tot = zeros16
            for r in range(L):
                tot = tot + hist[r, pl.ds(pl.multiple_of(k * L, L), L)]
            colsum[pl.ds(pl.multiple_of(k * L, L), L)] = tot
            padded = lax.shift_left(
                lax.shift_right_logical(tot + (L - 1), 4), 4
            )
            incl = plsc.cumsum(padded)
            colbase[pl.ds(pl.multiple_of(k * L, L), L)] = (
                incl - padded + bcast(carry)
            )
            return carry + jnp.sum(padded)

        lax.fori_loop(0, ARR // L, mkbase, 0)

        def mkcur(k, carry):
            acc = colbase[pl.ds(pl.multiple_of(k * L, L), L)]
            for r in range(L):
                row = hist[r, pl.ds(pl.multiple_of(k * L, L), L)]
                hist[r, pl.ds(pl.multiple_of(k * L, L), L)] = acc
                acc = acc + row
            return carry

        lax.fori_loop(0, ARR // L, mkcur, 0)

        def p2_chunk(b, g):
            def vec(i, carry2):
                v = ids[b, pl.ds(pl.multiple_of(i * L, L), L)]
                pos = bcast(g * ICH + i * L) + iota
                m, cl = tok_stats(v)
                slots = plsc.load_gather(hist, [iota, cl], mask=m)
                slots = jnp.minimum(slots, CAP - 1)
                plsc.store_scatter(sidx, [slots], v, mask=m)
                plsc.store_scatter(spos, [slots], pos, mask=m)
                plsc.addupdate_scatter(hist, [iota, cl], ones, mask=m)
                return carry2

            lax.fori_loop(0, ICH // L, vec, 0)

        stream_pass(p2_chunk)

        def lookup(ref, c):
            blk = lax.shift_right_logical(c, 4)
            v = _blk(ref, blk)
            oh = iota == bcast(c & (L - 1))
            return jnp.sum(jnp.where(oh, v, zeros16))

        col_sems = (sem_col0, sem_col1, sem_col2, sem_col3)
        out_sems = (sem_out0, sem_out1)

        def fire_col(c_rel, b):
            foff = jnp.minimum((lo + c_rel) * CW, last_off)
            pltpu.make_async_copy(
                tab_hbm.at[:, pl.ds(pl.multiple_of(foff, CW), CW)],
                colbuf.at[b], col_sems[b],
            ).start()

        def wait_col(b):
            pltpu.make_async_copy(
                tab_hbm.at[:, pl.ds(0, CW)], colbuf.at[b], col_sems[b]
            ).wait()

        def fire_out(sb):
            pltpu.make_async_copy(
                stage.at[sb], out_hbm.at[pbuf.at[sb]], out_sems[sb]
            ).start()

        def wait_out(sb):
            pltpu.make_async_copy(
                stage.at[sb], out_hbm.at[pbuf.at[sb]], out_sems[sb]
            ).wait()

        for sb in range(2):
            for r in range(GPB):
                pbuf[sb, pl.ds(r * L, L)] = bcast(total_rows) + iota
            fire_out(sb)
        for b in range(4):
            fire_col(b, b)

        dconsts = [jnp.full((L,), d, jnp.int32) for d in range(64)]

        def do_window(c_rel, gctr, cb):
            tot = lookup(colsum, c_rel)
            seg = lookup(colbase, c_rel)
            foff = jnp.minimum((lo + c_rel) * CW, last_off)
            wait_col(cb)
            cbv = jnp.full((L,), cb, jnp.int32)

            def group(k, gc):
                sb = lax.shift_right_logical(gc, 3) & 1
                rb = gc & (GPB - 1)
                s16 = pl.multiple_of(seg, L) + k * L
                idv = sidx[pl.ds(pl.multiple_of(s16, L), L)]
                psv = spos[pl.ds(pl.multiple_of(s16, L), L)]
                nreal = jnp.minimum(tot - k * L, L)
                m = iota < bcast(nreal)
                col = jnp.clip(idv - bcast(foff), 0, CW - 1)

                @pl.when(rb == 0)
                def _():
                    @pl.when(sb == 0)
                    def _():
                        wait_out(0)

                    @pl.when(sb == 1)
                    def _():
                        wait_out(1)

                sbv = bcast(sb)
                rows = bcast(rb * L) + iota
                for d in range(64):
                    vals = plsc.load_gather(colbuf, [cbv, dconsts[d], col])
                    plsc.store_scatter(stage, [sbv, rows, dconsts[d]], vals)
                plsc.store_scatter(
                    pbuf,
                    [sbv, rows],
                    jnp.where(m, psv, bcast(total_rows) + iota),
                )

                @pl.when(rb == GPB - 1)
                def _():
                    @pl.when(sb == 0)
                    def _():
                        fire_out(0)

                    @pl.when(sb == 1)
                    def _():
                        fire_out(1)

                return gc + 1

            ngroups = lax.shift_right_logical(tot + (L - 1), 4)
            return lax.fori_loop(0, ngroups, group, gctr)

        def colquad(gq, gctr):
            g4 = gq * 4
            for b in range(4):
                c_rel = g4 + b
                gctr = do_window(c_rel, gctr, b)
                fire_col(c_rel + 4, b)
            return gctr

        gctr = lax.fori_loop(0, (NCW - 1) // 4, colquad, 0)
        gctr = do_window(NCW - 1, gctr, (NCW - 1) & 3)
        for b in range(4):
            if b != (NCW - 1) & 3:
                wait_col(b)

        fsb = lax.shift_right_logical(gctr, 3) & 1

        @pl.when((gctr & (GPB - 1)) != 0)
        def _():
            @pl.when(fsb == 0)
            def _():
                fire_out(0)

            @pl.when(fsb == 1)
            def _():
                fire_out(1)

        for sb in range(2):
            wait_out(sb)

    return gather_kernel


def kernel(token_ids, table):
    batch, hist = token_ids.shape
    total = batch * hist
    vocab, dim = table.shape
    idx = token_ids.reshape(total).astype(jnp.int32)
    out = _make_kernel(total, vocab)(idx, table.T)
    return out[:total, :dim].reshape(batch, hist, dim)


# speedup vs baseline: 1.5819x; 1.5819x over previous
import functools

import jax
import jax.numpy as jnp
from jax import lax
from jax.experimental import pallas as pl
from jax.experimental.pallas import tpu as pltpu
from jax.experimental.pallas import tpu_sc as plsc

PDIM = 128
CHUNK = 128
NBUF = 5


@functools.lru_cache(maxsize=None)
def _make_kernel(total_rows):
    info = plsc.get_sparse_core_info()
    nc, ns = info.num_cores, info.num_subcores
    nw = nc * ns
    rows_per_w = total_rows // nw
    n_chunks = rows_per_w // CHUNK
    n_groups = n_chunks // NBUF
    assert rows_per_w * nw == total_rows
    assert n_chunks * CHUNK == rows_per_w
    assert n_groups * NBUF == n_chunks

    mesh = plsc.VectorSubcoreMesh(core_axis_name="c", subcore_axis_name="s")
    scratch = [pltpu.VMEM((rows_per_w,), jnp.int32)]
    scratch += [pltpu.VMEM((CHUNK, PDIM), jnp.float32) for _ in range(NBUF)]
    scratch += [pltpu.SemaphoreType.DMA for _ in range(NBUF)]

    @functools.partial(
        pl.kernel,
        mesh=mesh,
        out_type=jax.ShapeDtypeStruct((total_rows, PDIM), jnp.float32),
        scratch_types=scratch,
    )
    def gather_kernel(idx_hbm, table_hbm, out_hbm, idx_v, *rest):
        bufs = rest[:NBUF]
        sems = rest[NBUF:]
        wid = lax.axis_index("s") * nc + lax.axis_index("c")

        base = pl.multiple_of(wid * rows_per_w, rows_per_w)
        pltpu.sync_copy(idx_hbm.at[pl.ds(base, rows_per_w)], idx_v)

        def fire(j, b):
            start = pl.multiple_of(j * CHUNK, CHUNK)
            pltpu.make_async_copy(
                table_hbm.at[idx_v.at[pl.ds(start, CHUNK)]], bufs[b], sems[b]
            ).start()

        def drain(b):
            pltpu.make_async_copy(
                table_hbm.at[idx_v.at[pl.ds(0, CHUNK)]], bufs[b], sems[b]
            ).wait()

        def store(j, b):
            start = pl.multiple_of(base + j * CHUNK, CHUNK)
            pltpu.sync_copy(bufs[b], out_hbm.at[pl.ds(start, CHUNK)])

        for b in range(NBUF):
            fire(b, b)

        def body(g, carry):
            for b in range(NBUF):
                j = g * NBUF + b
                drain(b)
                store(j, b)
                fire(j + NBUF, b)
            return carry

        lax.fori_loop(0, n_groups - 1, body, 0)

        for b in range(NBUF):
            drain(b)
            store((n_groups - 1) * NBUF + b, b)

    return gather_kernel


def kernel(token_ids, table):
    batch, hist = token_ids.shape
    total = batch * hist
    dim = table.shape[1]
    idx = token_ids.reshape(total).astype(jnp.int32)
    table_p = jnp.pad(table, ((0, 0), (0, PDIM - dim)))
    out = _make_kernel(total)(idx, table_p)
    return out[:, :dim].reshape(batch, hist, dim)
